# baseline (speedup 1.0000x reference)
import functools

import jax
import jax.numpy as jnp
from jax import lax
from jax.experimental import pallas as pl
from jax.experimental.pallas import tpu as pltpu
from jax.experimental.pallas import tpu_sc as plsc

QB = 256
KB = 1024


def _fan_reduce(slices, op):
    acc = slices[0]
    for s in slices[1:]:
        acc = op(acc, s)
    return acc


def _wide_max(x):
    while x.shape[0] > 8:
        h = x.shape[0] // 8
        x = _fan_reduce([x[i * h:(i + 1) * h] for i in range(8)], jnp.maximum)
    return jnp.max(x, axis=0, keepdims=True)


def _first_argmax(s, m, t):
    n, qb = s.shape
    h = n // 8
    cand = _fan_reduce(
        [jnp.where(s[i * h:(i + 1) * h] == m,
                   lax.broadcasted_iota(jnp.int32, (h, qb), 0) + i * h,
                   t)
         for i in range(8)],
        jnp.minimum)
    while cand.shape[0] > 8:
        h = cand.shape[0] // 8
        cand = _fan_reduce([cand[i * h:(i + 1) * h] for i in range(8)],
                           jnp.minimum)
    return jnp.min(cand, axis=0)


def _kcols_body(x_ref, wkt_ref, kc_ref):
    kc_ref[0] = jnp.dot(x_ref[0], wkt_ref[...], preferred_element_type=jnp.float32)


def _main_body(x_ref, wq_ref, kc_ref, wvt_ref, v_ref, idx_ref):
    xb = x_ref[0]
    v_ref[0] = jnp.dot(xb, wvt_ref[...], preferred_element_type=jnp.float32)
    qt = lax.dot_general(wq_ref[...], xb, (((1,), (1,)), ((), ())),
                         preferred_element_type=jnp.float32)
    kc = kc_ref[0]
    t = kc.shape[0]
    s = lax.dot_general(kc.astype(jnp.bfloat16), qt.astype(jnp.bfloat16),
                        (((1,), (0,)), ((), ())),
                        preferred_element_type=jnp.float32)
    m = _wide_max(s)
    idx = _first_argmax(s, m, t)
    b = pl.program_id(0)
    idx_ref[0, 0, :] = idx + b * t


def _sc_gather(table, gidx):
    rows, d = table.shape
    nw = 32
    b_per_w = rows // nw
    ch = 64
    nch = b_per_w // ch
    mesh = plsc.VectorSubcoreMesh(core_axis_name="c", subcore_axis_name="s")

    @functools.partial(
        pl.kernel,
        mesh=mesh,
        out_type=jax.ShapeDtypeStruct((rows, d), jnp.float32),
        scratch_types=[
            pltpu.VMEM((ch,), jnp.int32),
            pltpu.VMEM((ch, d), jnp.float32),
            pltpu.VMEM((ch,), jnp.int32),
            pltpu.VMEM((ch, d), jnp.float32),
            pltpu.SemaphoreType.DMA,
            pltpu.SemaphoreType.DMA,
            pltpu.SemaphoreType.DMA,
            pltpu.SemaphoreType.DMA,
        ],
    )
    def k(table_hbm, idx_hbm, out_hbm, idx_a, rows_a, idx_b, rows_b,
          sg_a, sg_b, sw_a, sw_b):
        wid = lax.axis_index("s") * 2 + lax.axis_index("c")
        base = wid * b_per_w
        ib = [idx_a, idx_b]
        rb = [rows_a, rows_b]
        sg = [sg_a, sg_b]
        sw = [sw_a, sw_b]
        g = [None, None]
        w = [None, None]

        def issue_gather(c):
            j = c % 2
            pltpu.sync_copy(idx_hbm.at[pl.ds(base + c * ch, ch)], ib[j])
            g[j] = pltpu.async_copy(table_hbm.at[ib[j]], rb[j], sg[j])

        issue_gather(0)
        for c in range(nch):
            j = c % 2
            if c + 1 < nch:
                if c >= 1:
                    w[(c + 1) % 2].wait()
                issue_gather(c + 1)
            g[j].wait()
            w[j] = pltpu.async_copy(rb[j], out_hbm.at[pl.ds(base + c * ch, ch)],
                                    sw[j])
        for c in range(max(0, nch - 2), nch):
            w[c % 2].wait()

    return k(table, gidx)


def kernel(x, W_Q, W_K, W_V):
    B, T, D = x.shape
    nqb = T // QB
    wkt8 = jnp.pad(W_K.T, ((0, 0), (0, 8 - W_K.shape[0])))
    wq8 = jnp.pad(W_Q, ((0, 8 - W_Q.shape[0]), (0, 0)))
    wvt = W_V.T

    kcols = pl.pallas_call(
        _kcols_body,
        grid=(B, T // KB),
        in_specs=[
            pl.BlockSpec((1, KB, D), lambda b, t: (b, t, 0)),
            pl.BlockSpec((D, 8), lambda b, t: (0, 0)),
        ],
        out_specs=pl.BlockSpec((1, KB, 8), lambda b, t: (b, t, 0)),
        out_shape=jax.ShapeDtypeStruct((B, T, 8), jnp.float32),
    )(x, wkt8)

    v, idx = pl.pallas_call(
        _main_body,
        grid=(B, nqb),
        in_specs=[
            pl.BlockSpec((1, QB, D), lambda b, t: (b, t, 0)),
            pl.BlockSpec((8, D), lambda b, t: (0, 0)),
            pl.BlockSpec((1, T, 8), lambda b, t: (b, 0, 0)),
            pl.BlockSpec((D, D), lambda b, t: (0, 0)),
        ],
        out_specs=[
            pl.BlockSpec((1, QB, D), lambda b, t: (b, t, 0)),
            pl.BlockSpec((1, 1, QB), lambda b, t: (b * nqb + t, 0, 0)),
        ],
        out_shape=[
            jax.ShapeDtypeStruct((B, T, D), jnp.float32),
            jax.ShapeDtypeStruct((B * nqb, 1, QB), jnp.int32),
        ],
    )(x, wq8, kcols, wvt)

    table = v.reshape(B * T, D)
    gidx = idx.reshape(B * T)
    out = _sc_gather(table, gidx)
    return out.reshape(B, T, D)


# speedup vs baseline: 1.0541x; 1.0541x over previous
import functools

import jax
import jax.numpy as jnp
from jax import lax
from jax.experimental import pallas as pl
from jax.experimental.pallas import tpu as pltpu
from jax.experimental.pallas import tpu_sc as plsc

QB = 256
KB = 1024


def _kcols_body(x_ref, wkt_ref, kc_ref):
    kc = jnp.dot(x_ref[0], wkt_ref[...], preferred_element_type=jnp.float32)
    kc_ref[0] = kc.astype(jnp.bfloat16)


def _main_body(x_ref, wq_ref, kc_ref, wvt_ref, v_ref, idx_ref):
    xb = x_ref[0]
    v_ref[0] = jnp.dot(xb.astype(jnp.bfloat16), wvt_ref[...],
                       preferred_element_type=jnp.float32)
    qt = lax.dot_general(wq_ref[...], xb, (((1,), (1,)), ((), ())),
                         preferred_element_type=jnp.float32)
    kc = kc_ref[0]
    t = kc.shape[0]
    s = lax.dot_general(kc, qt.astype(jnp.bfloat16),
                        (((1,), (0,)), ((), ())),
                        preferred_element_type=jnp.float32)
    m = jnp.max(s, axis=0, keepdims=True)
    iota = lax.broadcasted_iota(jnp.int32, s.shape, 0)
    cand = jnp.where(s == m, iota, t)
    idx = jnp.min(cand, axis=0)
    b = pl.program_id(0)
    idx_ref[0, 0, :] = idx + b * t


def _sc_gather(table, gidx):
    rows, d = table.shape
    nw = 32
    b_per_w = rows // nw
    ch = 64
    nch = b_per_w // ch
    mesh = plsc.VectorSubcoreMesh(core_axis_name="c", subcore_axis_name="s")

    @functools.partial(
        pl.kernel,
        mesh=mesh,
        out_type=jax.ShapeDtypeStruct((rows, d), jnp.float32),
        scratch_types=[
            pltpu.VMEM((ch,), jnp.int32),
            pltpu.VMEM((ch, d), jnp.float32),
            pltpu.VMEM((ch,), jnp.int32),
            pltpu.VMEM((ch, d), jnp.float32),
            pltpu.SemaphoreType.DMA,
            pltpu.SemaphoreType.DMA,
            pltpu.SemaphoreType.DMA,
            pltpu.SemaphoreType.DMA,
        ],
    )
    def k(table_hbm, idx_hbm, out_hbm, idx_a, rows_a, idx_b, rows_b,
          sg_a, sg_b, sw_a, sw_b):
        wid = lax.axis_index("s") * 2 + lax.axis_index("c")
        base = wid * b_per_w
        ib = [idx_a, idx_b]
        rb = [rows_a, rows_b]
        sg = [sg_a, sg_b]
        sw = [sw_a, sw_b]
        g = [None, None]
        w = [None, None]

        def issue_gather(c):
            j = c % 2
            pltpu.sync_copy(idx_hbm.at[pl.ds(base + c * ch, ch)], ib[j])
            g[j] = pltpu.async_copy(table_hbm.at[ib[j]], rb[j], sg[j])

        issue_gather(0)
        for c in range(nch):
            j = c % 2
            if c + 1 < nch:
                if c >= 1:
                    w[(c + 1) % 2].wait()
                issue_gather(c + 1)
            g[j].wait()
            w[j] = pltpu.async_copy(rb[j], out_hbm.at[pl.ds(base + c * ch, ch)],
                                    sw[j])
        for c in range(max(0, nch - 2), nch):
            w[c % 2].wait()

    return k(table, gidx)


def kernel(x, W_Q, W_K, W_V):
    B, T, D = x.shape
    nqb = T // QB
    wkt8 = jnp.pad(W_K.T, ((0, 0), (0, 8 - W_K.shape[0])))
    wq8 = jnp.pad(W_Q, ((0, 8 - W_Q.shape[0]), (0, 0)))
    wvt = W_V.T.astype(jnp.bfloat16)

    kcols = pl.pallas_call(
        _kcols_body,
        grid=(B, T // KB),
        in_specs=[
            pl.BlockSpec((1, KB, D), lambda b, t: (b, t, 0)),
            pl.BlockSpec((D, 8), lambda b, t: (0, 0)),
        ],
        out_specs=pl.BlockSpec((1, KB, 8), lambda b, t: (b, t, 0)),
        out_shape=jax.ShapeDtypeStruct((B, T, 8), jnp.bfloat16),
    )(x, wkt8)

    v, idx = pl.pallas_call(
        _main_body,
        grid=(B, nqb),
        in_specs=[
            pl.BlockSpec((1, QB, D), lambda b, t: (b, t, 0)),
            pl.BlockSpec((8, D), lambda b, t: (0, 0)),
            pl.BlockSpec((1, T, 8), lambda b, t: (b, 0, 0)),
            pl.BlockSpec((D, D), lambda b, t: (0, 0)),
        ],
        out_specs=[
            pl.BlockSpec((1, QB, D), lambda b, t: (b, t, 0)),
            pl.BlockSpec((1, 1, QB), lambda b, t: (b * nqb + t, 0, 0)),
        ],
        out_shape=[
            jax.ShapeDtypeStruct((B, T, D), jnp.float32),
            jax.ShapeDtypeStruct((B * nqb, 1, QB), jnp.int32),
        ],
    )(x, wq8, kcols, wvt)

    table = v.reshape(B * T, D)
    gidx = idx.reshape(B * T)
    out = _sc_gather(table, gidx)
    return out.reshape(B, T, D)


# speedup vs baseline: 1.1223x; 1.0647x over previous
import functools

import jax
import jax.numpy as jnp
from jax import lax
from jax.experimental import pallas as pl
from jax.experimental.pallas import tpu as pltpu
from jax.experimental.pallas import tpu_sc as plsc

QB = 512
KB = 1024


def _kcols_body(x_ref, wkt_ref, kc_ref):
    kc = jnp.dot(x_ref[0], wkt_ref[...], preferred_element_type=jnp.float32)
    kc_ref[0] = kc.astype(jnp.bfloat16)


def _main_body(x_ref, wq_ref, kc_ref, wvt_ref, v_ref, idx_ref):
    xb = x_ref[0]
    v_ref[0] = jnp.dot(xb.astype(jnp.bfloat16), wvt_ref[...],
                       preferred_element_type=jnp.float32)
    qt = lax.dot_general(wq_ref[...], xb, (((1,), (1,)), ((), ())),
                         preferred_element_type=jnp.float32)
    kc = kc_ref[0]
    t = kc.shape[0]
    s = lax.dot_general(kc, qt.astype(jnp.bfloat16),
                        (((1,), (0,)), ((), ())),
                        preferred_element_type=jnp.float32)
    m = jnp.max(s, axis=0, keepdims=True)
    iota = lax.broadcasted_iota(jnp.int32, s.shape, 0)
    cand = jnp.where(s == m, iota, t)
    idx = jnp.min(cand, axis=0)
    b = pl.program_id(0)
    idx_ref[0, 0, :] = idx + b * t


def _sc_gather(table, gidx):
    rows, d = table.shape
    nw = 32
    b_per_w = rows // nw
    ch = 64
    nch = b_per_w // ch
    mesh = plsc.VectorSubcoreMesh(core_axis_name="c", subcore_axis_name="s")

    @functools.partial(
        pl.kernel,
        mesh=mesh,
        out_type=jax.ShapeDtypeStruct((rows, d), jnp.float32),
        scratch_types=[
            pltpu.VMEM((ch,), jnp.int32),
            pltpu.VMEM((ch, d), jnp.float32),
            pltpu.VMEM((ch,), jnp.int32),
            pltpu.VMEM((ch, d), jnp.float32),
            pltpu.SemaphoreType.DMA,
            pltpu.SemaphoreType.DMA,
            pltpu.SemaphoreType.DMA,
            pltpu.SemaphoreType.DMA,
        ],
    )
    def k(table_hbm, idx_hbm, out_hbm, idx_a, rows_a, idx_b, rows_b,
          sg_a, sg_b, sw_a, sw_b):
        wid = lax.axis_index("s") * 2 + lax.axis_index("c")
        base = wid * b_per_w
        ib = [idx_a, idx_b]
        rb = [rows_a, rows_b]
        sg = [sg_a, sg_b]
        sw = [sw_a, sw_b]
        g = [None, None]
        w = [None, None]

        def issue_gather(c):
            j = c % 2
            pltpu.sync_copy(idx_hbm.at[pl.ds(base + c * ch, ch)], ib[j])
            g[j] = pltpu.async_copy(table_hbm.at[ib[j]], rb[j], sg[j])

        issue_gather(0)
        for c in range(nch):
            j = c % 2
            if c + 1 < nch:
                if c >= 1:
                    w[(c + 1) % 2].wait()
                issue_gather(c + 1)
            g[j].wait()
            w[j] = pltpu.async_copy(rb[j], out_hbm.at[pl.ds(base + c * ch, ch)],
                                    sw[j])
        for c in range(max(0, nch - 2), nch):
            w[c % 2].wait()

    return k(table, gidx)


def kernel(x, W_Q, W_K, W_V):
    B, T, D = x.shape
    nqb = T // QB
    wkt8 = jnp.pad(W_K.T, ((0, 0), (0, 8 - W_K.shape[0])))
    wq8 = jnp.pad(W_Q, ((0, 8 - W_Q.shape[0]), (0, 0)))
    wvt = W_V.T.astype(jnp.bfloat16)

    kcols = pl.pallas_call(
        _kcols_body,
        grid=(B, T // KB),
        in_specs=[
            pl.BlockSpec((1, KB, D), lambda b, t: (b, t, 0)),
            pl.BlockSpec((D, 8), lambda b, t: (0, 0)),
        ],
        out_specs=pl.BlockSpec((1, KB, 8), lambda b, t: (b, t, 0)),
        out_shape=jax.ShapeDtypeStruct((B, T, 8), jnp.bfloat16),
    )(x, wkt8)

    v, idx = pl.pallas_call(
        _main_body,
        grid=(B, nqb),
        in_specs=[
            pl.BlockSpec((1, QB, D), lambda b, t: (b, t, 0)),
            pl.BlockSpec((8, D), lambda b, t: (0, 0)),
            pl.BlockSpec((1, T, 8), lambda b, t: (b, 0, 0)),
            pl.BlockSpec((D, D), lambda b, t: (0, 0)),
        ],
        out_specs=[
            pl.BlockSpec((1, QB, D), lambda b, t: (b, t, 0)),
            pl.BlockSpec((1, 1, QB), lambda b, t: (b * nqb + t, 0, 0)),
        ],
        out_shape=[
            jax.ShapeDtypeStruct((B, T, D), jnp.float32),
            jax.ShapeDtypeStruct((B * nqb, 1, QB), jnp.int32),
        ],
    )(x, wq8, kcols, wvt)

    table = v.reshape(B * T, D)
    gidx = idx.reshape(B * T)
    out = _sc_gather(table, gidx)
    return out.reshape(B, T, D)


# speedup vs baseline: 1.1545x; 1.0287x over previous
import functools

import jax
import jax.numpy as jnp
from jax import lax
from jax.experimental import pallas as pl
from jax.experimental.pallas import tpu as pltpu
from jax.experimental.pallas import tpu_sc as plsc

QB = 1024
KB = 1024


def _kcols_body(x_ref, wkt_ref, kc_ref):
    kc = jnp.dot(x_ref[0], wkt_ref[...], preferred_element_type=jnp.float32)
    kc_ref[0] = kc.astype(jnp.bfloat16)


def _main_body(x_ref, wq_ref, kc_ref, wvt_ref, v_ref, idx_ref):
    xb = x_ref[0]
    v_ref[0] = jnp.dot(xb.astype(jnp.bfloat16), wvt_ref[...],
                       preferred_element_type=jnp.float32)
    qt = lax.dot_general(wq_ref[...], xb, (((1,), (1,)), ((), ())),
                         preferred_element_type=jnp.float32)
    kc = kc_ref[0]
    t = kc.shape[0]
    s = lax.dot_general(kc, qt.astype(jnp.bfloat16),
                        (((1,), (0,)), ((), ())),
                        preferred_element_type=jnp.float32)
    m = jnp.max(s, axis=0, keepdims=True)
    iota = lax.broadcasted_iota(jnp.int32, s.shape, 0)
    cand = jnp.where(s == m, iota, t)
    idx = jnp.min(cand, axis=0)
    b = pl.program_id(0)
    idx_ref[0, 0, :] = idx + b * t


def _sc_gather(table, gidx):
    rows, d = table.shape
    nw = 32
    b_per_w = rows // nw
    ch = 64
    nch = b_per_w // ch
    mesh = plsc.VectorSubcoreMesh(core_axis_name="c", subcore_axis_name="s")

    @functools.partial(
        pl.kernel,
        mesh=mesh,
        out_type=jax.ShapeDtypeStruct((rows, d), jnp.float32),
        scratch_types=[
            pltpu.VMEM((ch,), jnp.int32),
            pltpu.VMEM((ch, d), jnp.float32),
            pltpu.VMEM((ch,), jnp.int32),
            pltpu.VMEM((ch, d), jnp.float32),
            pltpu.SemaphoreType.DMA,
            pltpu.SemaphoreType.DMA,
            pltpu.SemaphoreType.DMA,
            pltpu.SemaphoreType.DMA,
        ],
    )
    def k(table_hbm, idx_hbm, out_hbm, idx_a, rows_a, idx_b, rows_b,
          sg_a, sg_b, sw_a, sw_b):
        wid = lax.axis_index("s") * 2 + lax.axis_index("c")
        base = wid * b_per_w
        ib = [idx_a, idx_b]
        rb = [rows_a, rows_b]
        sg = [sg_a, sg_b]
        sw = [sw_a, sw_b]
        g = [None, None]
        w = [None, None]

        def issue_gather(c):
            j = c % 2
            pltpu.sync_copy(idx_hbm.at[pl.ds(base + c * ch, ch)], ib[j])
            g[j] = pltpu.async_copy(table_hbm.at[ib[j]], rb[j], sg[j])

        issue_gather(0)
        for c in range(nch):
            j = c % 2
            if c + 1 < nch:
                if c >= 1:
                    w[(c + 1) % 2].wait()
                issue_gather(c + 1)
            g[j].wait()
            w[j] = pltpu.async_copy(rb[j], out_hbm.at[pl.ds(base + c * ch, ch)],
                                    sw[j])
        for c in range(max(0, nch - 2), nch):
            w[c % 2].wait()

    return k(table, gidx)


def kernel(x, W_Q, W_K, W_V):
    B, T, D = x.shape
    nqb = T // QB
    wkt8 = jnp.pad(W_K.T, ((0, 0), (0, 8 - W_K.shape[0])))
    wq8 = jnp.pad(W_Q, ((0, 8 - W_Q.shape[0]), (0, 0)))
    wvt = W_V.T.astype(jnp.bfloat16)

    kcols = pl.pallas_call(
        _kcols_body,
        grid=(B, T // KB),
        in_specs=[
            pl.BlockSpec((1, KB, D), lambda b, t: (b, t, 0)),
            pl.BlockSpec((D, 8), lambda b, t: (0, 0)),
        ],
        out_specs=pl.BlockSpec((1, KB, 8), lambda b, t: (b, t, 0)),
        out_shape=jax.ShapeDtypeStruct((B, T, 8), jnp.bfloat16),
    )(x, wkt8)

    v, idx = pl.pallas_call(
        _main_body,
        grid=(B, nqb),
        in_specs=[
            pl.BlockSpec((1, QB, D), lambda b, t: (b, t, 0)),
            pl.BlockSpec((8, D), lambda b, t: (0, 0)),
            pl.BlockSpec((1, T, 8), lambda b, t: (b, 0, 0)),
            pl.BlockSpec((D, D), lambda b, t: (0, 0)),
        ],
        out_specs=[
            pl.BlockSpec((1, QB, D), lambda b, t: (b, t, 0)),
            pl.BlockSpec((1, 1, QB), lambda b, t: (b * nqb + t, 0, 0)),
        ],
        out_shape=[
            jax.ShapeDtypeStruct((B, T, D), jnp.float32),
            jax.ShapeDtypeStruct((B * nqb, 1, QB), jnp.int32),
        ],
    )(x, wq8, kcols, wvt)

    table = v.reshape(B * T, D)
    gidx = idx.reshape(B * T)
    out = _sc_gather(table, gidx)
    return out.reshape(B, T, D)


# speedup vs baseline: 1.1595x; 1.0044x over previous
import functools

import jax
import jax.numpy as jnp
from jax import lax
from jax.experimental import pallas as pl
from jax.experimental.pallas import tpu as pltpu
from jax.experimental.pallas import tpu_sc as plsc

QB = 2048
KB = 1024


def _kcols_body(x_ref, wkt_ref, kc_ref):
    kc = jnp.dot(x_ref[0], wkt_ref[...], preferred_element_type=jnp.float32)
    kc_ref[0] = kc.astype(jnp.bfloat16)


def _main_body(x_ref, wq_ref, kc_ref, wvt_ref, v_ref, idx_ref):
    xb = x_ref[0]
    v_ref[0] = jnp.dot(xb.astype(jnp.bfloat16), wvt_ref[...],
                       preferred_element_type=jnp.float32)
    qt = lax.dot_general(wq_ref[...], xb, (((1,), (1,)), ((), ())),
                         preferred_element_type=jnp.float32)
    kc = kc_ref[0]
    t = kc.shape[0]
    s = lax.dot_general(kc, qt.astype(jnp.bfloat16),
                        (((1,), (0,)), ((), ())),
                        preferred_element_type=jnp.float32)
    m = jnp.max(s, axis=0, keepdims=True)
    iota = lax.broadcasted_iota(jnp.int32, s.shape, 0)
    cand = jnp.where(s == m, iota, t)
    idx = jnp.min(cand, axis=0)
    b = pl.program_id(0)
    idx_ref[0, 0, :] = idx + b * t


def _sc_gather(table, gidx):
    rows, d = table.shape
    nw = 32
    b_per_w = rows // nw
    ch = 64
    nch = b_per_w // ch
    mesh = plsc.VectorSubcoreMesh(core_axis_name="c", subcore_axis_name="s")

    @functools.partial(
        pl.kernel,
        mesh=mesh,
        out_type=jax.ShapeDtypeStruct((rows, d), jnp.float32),
        scratch_types=[
            pltpu.VMEM((ch,), jnp.int32),
            pltpu.VMEM((ch, d), jnp.float32),
            pltpu.VMEM((ch,), jnp.int32),
            pltpu.VMEM((ch, d), jnp.float32),
            pltpu.SemaphoreType.DMA,
            pltpu.SemaphoreType.DMA,
            pltpu.SemaphoreType.DMA,
            pltpu.SemaphoreType.DMA,
        ],
    )
    def k(table_hbm, idx_hbm, out_hbm, idx_a, rows_a, idx_b, rows_b,
          sg_a, sg_b, sw_a, sw_b):
        wid = lax.axis_index("s") * 2 + lax.axis_index("c")
        base = wid * b_per_w
        ib = [idx_a, idx_b]
        rb = [rows_a, rows_b]
        sg = [sg_a, sg_b]
        sw = [sw_a, sw_b]
        g = [None, None]
        w = [None, None]

        def issue_gather(c):
            j = c % 2
            pltpu.sync_copy(idx_hbm.at[pl.ds(base + c * ch, ch)], ib[j])
            g[j] = pltpu.async_copy(table_hbm.at[ib[j]], rb[j], sg[j])

        issue_gather(0)
        for c in range(nch):
            j = c % 2
            if c + 1 < nch:
                if c >= 1:
                    w[(c + 1) % 2].wait()
                issue_gather(c + 1)
            g[j].wait()
            w[j] = pltpu.async_copy(rb[j], out_hbm.at[pl.ds(base + c * ch, ch)],
                                    sw[j])
        for c in range(max(0, nch - 2), nch):
            w[c % 2].wait()

    return k(table, gidx)


def kernel(x, W_Q, W_K, W_V):
    B, T, D = x.shape
    nqb = T // QB
    wkt8 = jnp.pad(W_K.T, ((0, 0), (0, 8 - W_K.shape[0])))
    wq8 = jnp.pad(W_Q, ((0, 8 - W_Q.shape[0]), (0, 0)))
    wvt = W_V.T.astype(jnp.bfloat16)

    kcols = pl.pallas_call(
        _kcols_body,
        grid=(B, T // KB),
        in_specs=[
            pl.BlockSpec((1, KB, D), lambda b, t: (b, t, 0)),
            pl.BlockSpec((D, 8), lambda b, t: (0, 0)),
        ],
        out_specs=pl.BlockSpec((1, KB, 8), lambda b, t: (b, t, 0)),
        out_shape=jax.ShapeDtypeStruct((B, T, 8), jnp.bfloat16),
    )(x, wkt8)

    v, idx = pl.pallas_call(
        _main_body,
        grid=(B, nqb),
        in_specs=[
            pl.BlockSpec((1, QB, D), lambda b, t: (b, t, 0)),
            pl.BlockSpec((8, D), lambda b, t: (0, 0)),
            pl.BlockSpec((1, T, 8), lambda b, t: (b, 0, 0)),
            pl.BlockSpec((D, D), lambda b, t: (0, 0)),
        ],
        out_specs=[
            pl.BlockSpec((1, QB, D), lambda b, t: (b, t, 0)),
            pl.BlockSpec((1, 1, QB), lambda b, t: (b * nqb + t, 0, 0)),
        ],
        out_shape=[
            jax.ShapeDtypeStruct((B, T, D), jnp.float32),
            jax.ShapeDtypeStruct((B * nqb, 1, QB), jnp.int32),
        ],
    )(x, wq8, kcols, wvt)

    table = v.reshape(B * T, D)
    gidx = idx.reshape(B * T)
    out = _sc_gather(table, gidx)
    return out.reshape(B, T, D)
